# initial kernel scaffold (unmeasured)
import jax
import jax.numpy as jnp
from jax import lax
from jax.experimental import pallas as pl
from jax.experimental.pallas import tpu as pltpu

N_DEV = 4
SQ = 256
D = 1024
HQ = 8
HKV = 2
DH = 128
SCALE = 0.08838834764831843

PKT_W = D + 128
M_COL = D
L_COL = D + HQ

PRECISION = lax.Precision.HIGHEST


def kernel(x, Wq, Wo, K_ext, V_ext):
    skv_local = K_ext.shape[1]

    def body(x_ref, wq_ref, wo_ref, k_ref, v_ref, out_ref,
             pkt_ref, recv_ref, send_sems, recv_sems, copy_sem):
        my_pos = lax.axis_index("i")

        barrier_sem = pltpu.get_barrier_semaphore()
        for d in range(1, N_DEV):
            pl.semaphore_signal(
                barrier_sem, inc=1,
                device_id=((my_pos + d) % N_DEV,),
                device_id_type=pl.DeviceIdType.MESH,
            )
        pl.semaphore_wait(barrier_sem, N_DEV - 1)

        q = jnp.dot(x_ref[0], wq_ref[...], precision=PRECISION,
                    preferred_element_type=jnp.float32)
        for h in range(HQ):
            kv = h // (HQ // HKV)
            q_h = q[:, h * DH:(h + 1) * DH]
            k_h = k_ref[0, :, kv, :]
            v_h = v_ref[0, :, kv, :]
            s = lax.dot_general(
                q_h, k_h, (((1,), (1,)), ((), ())),
                precision=PRECISION, preferred_element_type=jnp.float32,
            ) * SCALE
            m_h = jnp.max(s, axis=1, keepdims=True)
            p = jnp.exp(s - m_h)
            l_h = jnp.sum(p, axis=1, keepdims=True)
            o_h = jnp.dot(p, v_h, precision=PRECISION,
                          preferred_element_type=jnp.float32)
            pkt_ref[:, h * DH:(h + 1) * DH] = o_h
            pkt_ref[:, M_COL + h:M_COL + h + 1] = m_h
            pkt_ref[:, L_COL + h:L_COL + h + 1] = l_h

        local_cp = pltpu.make_async_copy(
            pkt_ref, recv_ref.at[my_pos], copy_sem)
        local_cp.start()

        sends = []
        for d in range(1, N_DEV):
            peer = (my_pos + d) % N_DEV
            rdma = pltpu.make_async_remote_copy(
                src_ref=pkt_ref,
                dst_ref=recv_ref.at[my_pos],
                send_sem=send_sems.at[d - 1],
                recv_sem=recv_sems.at[my_pos],
                device_id=(peer,),
                device_id_type=pl.DeviceIdType.MESH,
            )
            rdma.start()
            sends.append(rdma)

        local_cp.wait()
        for rdma in sends:
            rdma.wait_send()
        for d in range(1, N_DEV):
            src = (my_pos + d) % N_DEV
            recv = pltpu.make_async_remote_copy(
                src_ref=pkt_ref,
                dst_ref=recv_ref.at[src],
                send_sem=send_sems.at[d - 1],
                recv_sem=recv_sems.at[src],
                device_id=(src,),
                device_id_type=pl.DeviceIdType.MESH,
            )
            recv.wait_recv()

        attn_heads = []
        for h in range(HQ):
            c0 = h * DH
            ms = [recv_ref[s, :, M_COL + h:M_COL + h + 1] for s in range(N_DEV)]
            m_glob = ms[0]
            for m_s in ms[1:]:
                m_glob = jnp.maximum(m_glob, m_s)
            o_acc = jnp.zeros((SQ, DH), jnp.float32)
            l_acc = jnp.zeros((SQ, 1), jnp.float32)
            for s in range(N_DEV):
                w = jnp.exp(ms[s] - m_glob)
                o_acc = o_acc + recv_ref[s, :, c0:c0 + DH] * w
                l_acc = l_acc + recv_ref[s, :, L_COL + h:L_COL + h + 1] * w
            attn_heads.append(o_acc / l_acc)
        attn = jnp.concatenate(attn_heads, axis=1)

        out_ref[0] = jnp.dot(attn, wo_ref[...], precision=PRECISION,
                             preferred_element_type=jnp.float32)

    return pl.pallas_call(
        body,
        out_shape=jax.ShapeDtypeStruct((1, SQ, D), jnp.float32),
        in_specs=[pl.BlockSpec(memory_space=pltpu.VMEM)] * 5,
        out_specs=pl.BlockSpec(memory_space=pltpu.VMEM),
        scratch_shapes=[
            pltpu.VMEM((SQ, PKT_W), jnp.float32),
            pltpu.VMEM((N_DEV, SQ, PKT_W), jnp.float32),
            pltpu.SemaphoreType.DMA((N_DEV - 1,)),
            pltpu.SemaphoreType.DMA((N_DEV,)),
            pltpu.SemaphoreType.DMA,
        ],
        compiler_params=pltpu.CompilerParams(collective_id=0),
    )(x, Wq, Wo, K_ext, V_ext)


# baseline (device time: 83398 ns/iter reference)
import jax
import jax.numpy as jnp
from jax import lax
from jax.experimental import pallas as pl
from jax.experimental.pallas import tpu as pltpu

N_DEV = 4
SQ = 256
D = 1024
HQ = 8
HKV = 2
GRP = HQ // HKV
DH = 128
ROWS = HQ * SQ
SCALE = 0.08838834764831843

PKT_W = DH + 2

PRECISION = lax.Precision.DEFAULT


def kernel(x, Wq, Wo, K_ext, V_ext):
    def body(x_ref, wq_ref, wo_ref, k_ref, v_ref, out_ref,
             pkt_ref, recv_ref, send_sems, recv_sems, copy_sem):
        my_pos = lax.axis_index("i")

        barrier_sem = pltpu.get_barrier_semaphore()
        for d in range(1, N_DEV):
            pl.semaphore_signal(
                barrier_sem, inc=1,
                device_id=((my_pos + d) % N_DEV,),
                device_id_type=pl.DeviceIdType.MESH,
            )
        pl.semaphore_wait(barrier_sem, N_DEV - 1)

        q = jnp.dot(x_ref[0], wq_ref[...], precision=PRECISION,
                    preferred_element_type=jnp.float32)
        for g in range(HKV):
            q_grp = q[:, g * GRP * DH:(g + 1) * GRP * DH]
            q_rows = q_grp.reshape(SQ, GRP, DH).swapaxes(0, 1).reshape(
                GRP * SQ, DH)
            k_g = k_ref[0, :, g, :]
            v_g = v_ref[0, :, g, :]
            s = lax.dot_general(
                q_rows, k_g, (((1,), (1,)), ((), ())),
                precision=PRECISION, preferred_element_type=jnp.float32,
            ) * SCALE
            m_g = jnp.max(s, axis=1, keepdims=True)
            p = jnp.exp(s - m_g)
            l_g = jnp.sum(p, axis=1, keepdims=True)
            o_g = jnp.dot(p, v_g, precision=PRECISION,
                          preferred_element_type=jnp.float32)
            r0 = g * GRP * SQ
            pkt_ref[r0:r0 + GRP * SQ, 0:DH] = o_g
            pkt_ref[r0:r0 + GRP * SQ, DH:DH + 1] = m_g
            pkt_ref[r0:r0 + GRP * SQ, DH + 1:DH + 2] = l_g

        local_cp = pltpu.make_async_copy(
            pkt_ref, recv_ref.at[my_pos], copy_sem)
        local_cp.start()

        sends = []
        for d in range(1, N_DEV):
            peer = (my_pos + d) % N_DEV
            rdma = pltpu.make_async_remote_copy(
                src_ref=pkt_ref,
                dst_ref=recv_ref.at[my_pos],
                send_sem=send_sems.at[d - 1],
                recv_sem=recv_sems.at[my_pos],
                device_id=(peer,),
                device_id_type=pl.DeviceIdType.MESH,
            )
            rdma.start()
            sends.append(rdma)

        local_cp.wait()
        for rdma in sends:
            rdma.wait_send()
        for d in range(1, N_DEV):
            src = (my_pos + d) % N_DEV
            recv = pltpu.make_async_remote_copy(
                src_ref=pkt_ref,
                dst_ref=recv_ref.at[src],
                send_sem=send_sems.at[d - 1],
                recv_sem=recv_sems.at[src],
                device_id=(src,),
                device_id_type=pl.DeviceIdType.MESH,
            )
            recv.wait_recv()

        ms = [recv_ref[s, :, DH:DH + 1] for s in range(N_DEV)]
        m_glob = ms[0]
        for m_s in ms[1:]:
            m_glob = jnp.maximum(m_glob, m_s)
        o_acc = jnp.zeros((ROWS, DH), jnp.float32)
        l_acc = jnp.zeros((ROWS, 1), jnp.float32)
        for s in range(N_DEV):
            w = jnp.exp(ms[s] - m_glob)
            o_acc = o_acc + recv_ref[s, :, 0:DH] * w
            l_acc = l_acc + recv_ref[s, :, DH + 1:DH + 2] * w
        attn_rows = o_acc / l_acc

        attn = attn_rows.reshape(HQ, SQ, DH).swapaxes(0, 1).reshape(SQ, D)
        out_ref[0] = jnp.dot(attn, wo_ref[...], precision=PRECISION,
                             preferred_element_type=jnp.float32)

    return pl.pallas_call(
        body,
        out_shape=jax.ShapeDtypeStruct((1, SQ, D), jnp.float32),
        in_specs=[pl.BlockSpec(memory_space=pltpu.VMEM)] * 5,
        out_specs=pl.BlockSpec(memory_space=pltpu.VMEM),
        scratch_shapes=[
            pltpu.VMEM((ROWS, PKT_W), jnp.float32),
            pltpu.VMEM((N_DEV, ROWS, PKT_W), jnp.float32),
            pltpu.SemaphoreType.DMA((N_DEV - 1,)),
            pltpu.SemaphoreType.DMA((N_DEV,)),
            pltpu.SemaphoreType.DMA,
        ],
        compiler_params=pltpu.CompilerParams(
            collective_id=0,
            vmem_limit_bytes=100 * 1024 * 1024,
        ),
    )(x, Wq, Wo, K_ext, V_ext)


# device time: 73442 ns/iter; 1.1356x vs baseline; 1.1356x over previous
import jax
import jax.numpy as jnp
from jax import lax
from jax.experimental import pallas as pl
from jax.experimental.pallas import tpu as pltpu

N_DEV = 4
SQ = 256
D = 1024
HQ = 8
HKV = 2
GRP = HQ // HKV
DH = 128
GROWS = GRP * SQ
ROWS = HKV * GROWS
SCALE = 0.08838834764831843

PKT_W = DH + 2

PRECISION = lax.Precision.DEFAULT


def kernel(x, Wq, Wo, K_ext, V_ext):
    def body(x_ref, wq_ref, wo_ref, k_ref, v_ref, out_ref,
             pkt_ref, recv_ref, send_sems, recv_sems, copy_sems):
        my_pos = lax.axis_index("i")

        barrier_sem = pltpu.get_barrier_semaphore()
        for d in range(1, N_DEV):
            pl.semaphore_signal(
                barrier_sem, inc=1,
                device_id=((my_pos + d) % N_DEV,),
                device_id_type=pl.DeviceIdType.MESH,
            )
        pl.semaphore_wait(barrier_sem, N_DEV - 1)

        q = jnp.dot(x_ref[0], wq_ref[...], precision=PRECISION,
                    preferred_element_type=jnp.float32)

        sends = []
        local_cps = []
        for g in range(HKV):
            q_grp = q[:, g * GRP * DH:(g + 1) * GRP * DH]
            q_rows = q_grp.reshape(SQ, GRP, DH).swapaxes(0, 1).reshape(
                GROWS, DH)
            k_g = k_ref[0, :, g, :]
            v_g = v_ref[0, :, g, :]
            s = lax.dot_general(
                q_rows, k_g, (((1,), (1,)), ((), ())),
                precision=PRECISION, preferred_element_type=jnp.float32,
            ) * SCALE
            m_g = jnp.max(s, axis=1, keepdims=True)
            p = jnp.exp(s - m_g)
            l_g = jnp.sum(p, axis=1, keepdims=True)
            o_g = jnp.dot(p, v_g, precision=PRECISION,
                          preferred_element_type=jnp.float32)
            r0 = g * GROWS
            pkt_ref[r0:r0 + GROWS, 0:DH] = o_g
            pkt_ref[r0:r0 + GROWS, DH:DH + 1] = m_g
            pkt_ref[r0:r0 + GROWS, DH + 1:DH + 2] = l_g

            cp = pltpu.make_async_copy(
                pkt_ref.at[pl.ds(r0, GROWS)],
                recv_ref.at[my_pos, pl.ds(r0, GROWS)],
                copy_sems.at[g])
            cp.start()
            local_cps.append(cp)
            for d in range(1, N_DEV):
                peer = (my_pos + d) % N_DEV
                rdma = pltpu.make_async_remote_copy(
                    src_ref=pkt_ref.at[pl.ds(r0, GROWS)],
                    dst_ref=recv_ref.at[my_pos, pl.ds(r0, GROWS)],
                    send_sem=send_sems.at[g, d - 1],
                    recv_sem=recv_sems.at[g, my_pos],
                    device_id=(peer,),
                    device_id_type=pl.DeviceIdType.MESH,
                )
                rdma.start()
                sends.append(rdma)

        out_acc = jnp.zeros((SQ, D), jnp.float32)
        for g in range(HKV):
            r0 = g * GROWS
            local_cps[g].wait()
            for d in range(1, N_DEV):
                src = (my_pos + d) % N_DEV
                recv = pltpu.make_async_remote_copy(
                    src_ref=pkt_ref.at[pl.ds(r0, GROWS)],
                    dst_ref=recv_ref.at[src, pl.ds(r0, GROWS)],
                    send_sem=send_sems.at[g, d - 1],
                    recv_sem=recv_sems.at[g, src],
                    device_id=(src,),
                    device_id_type=pl.DeviceIdType.MESH,
                )
                recv.wait_recv()

            ms = [recv_ref[s, r0:r0 + GROWS, DH:DH + 1] for s in range(N_DEV)]
            m_glob = ms[0]
            for m_s in ms[1:]:
                m_glob = jnp.maximum(m_glob, m_s)
            o_acc = jnp.zeros((GROWS, DH), jnp.float32)
            l_acc = jnp.zeros((GROWS, 1), jnp.float32)
            for s in range(N_DEV):
                w = jnp.exp(ms[s] - m_glob)
                o_acc = o_acc + recv_ref[s, r0:r0 + GROWS, 0:DH] * w
                l_acc = l_acc + recv_ref[s, r0:r0 + GROWS, DH + 1:DH + 2] * w
            attn_g = o_acc / l_acc

            for hl in range(GRP):
                h = g * GRP + hl
                out_acc = out_acc + jnp.dot(
                    attn_g[hl * SQ:(hl + 1) * SQ, :],
                    wo_ref[h * DH:(h + 1) * DH, :],
                    precision=PRECISION,
                    preferred_element_type=jnp.float32)

        out_ref[0] = out_acc
        for rdma in sends:
            rdma.wait_send()

    return pl.pallas_call(
        body,
        out_shape=jax.ShapeDtypeStruct((1, SQ, D), jnp.float32),
        in_specs=[pl.BlockSpec(memory_space=pltpu.VMEM)] * 5,
        out_specs=pl.BlockSpec(memory_space=pltpu.VMEM),
        scratch_shapes=[
            pltpu.VMEM((ROWS, PKT_W), jnp.float32),
            pltpu.VMEM((N_DEV, ROWS, PKT_W), jnp.float32),
            pltpu.SemaphoreType.DMA((HKV, N_DEV - 1)),
            pltpu.SemaphoreType.DMA((HKV, N_DEV)),
            pltpu.SemaphoreType.DMA((HKV,)),
        ],
        compiler_params=pltpu.CompilerParams(
            collective_id=0,
            vmem_limit_bytes=100 * 1024 * 1024,
        ),
    )(x, Wq, Wo, K_ext, V_ext)
